# baseline (speedup 1.0000x reference)
import functools

import jax
import jax.numpy as jnp
from jax import lax
from jax.experimental import pallas as pl
from jax.experimental.pallas import tpu as pltpu
from jax.experimental.pallas import tpu_sc as plsc

N = 10000
FD = 256
D1 = 512
D2 = 256
E = 160000
BT = 32
PN = 512

NC = 2
NS = 16
NW = NC * NS
HALF = N // NC
NPAD = N + 8
ES = E // NS
NCH = ES // 128
TAILK = ES - NCH * 128


def _mesh():
    return plsc.VectorSubcoreMesh(core_axis_name="c", subcore_axis_name="s")


def _deg_sc(dst1, dst2):
    @functools.partial(
        pl.kernel,
        out_type=[jax.ShapeDtypeStruct((NPAD, 256), jnp.float32)] * 2,
        mesh=_mesh(),
        scratch_types=[
            pltpu.VMEM((ES,), jnp.int32),
            pltpu.VMEM((128,), jnp.int32),
            pltpu.VMEM((TAILK,), jnp.int32),
            pltpu.VMEM((128, 256), jnp.float32),
            pltpu.VMEM((128, 256), jnp.float32),
        ],
    )
    def k(d1_h, d2_h, o1_h, o2_h, dbuf, locb, locbt, zrows, ones):
        cid = lax.axis_index("c")
        sid = lax.axis_index("s")
        base = cid * HALF
        z16 = jnp.zeros((16,), jnp.float32)
        o16 = jnp.full((16,), 1.0, jnp.float32)

        def fill(t, _):
            for h in range(16):
                zrows[t, pl.ds(h * 16, 16)] = z16
                ones[t, pl.ds(h * 16, 16)] = o16
            return _
        lax.fori_loop(0, 128, fill, None)

        for g in range(2):
            d_h = (d1_h, d2_h)[g]
            o_h = (o1_h, o2_h)[g]

            for tt, cnt in ((0, 128), (128, 128), (256, 56)):
                pltpu.sync_copy(zrows.at[pl.ds(0, cnt)],
                                o_h.at[pl.ds(base + sid * 312 + tt, cnt)])

            @pl.when(sid == 0)
            def _():
                pltpu.sync_copy(zrows.at[pl.ds(0, 8)],
                                o_h.at[pl.ds(base + 4992, 8)])
            plsc.subcore_barrier()

            eoff = pl.multiple_of(sid * ES, 8)
            pltpu.sync_copy(d_h.at[pl.ds(eoff, ES)], dbuf)

            def remap(coff, kk, lb):
                for j in range(kk // 16):
                    v = dbuf[pl.ds(coff + j * 16, 16)]
                    ok = (v >= base) & (v < base + HALF)
                    lb[pl.ds(j * 16, 16)] = jnp.where(ok, v, N)

            def body(i, _):
                remap(i * 128, 128, locb)
                pltpu.sync_copy(ones, o_h.at[locb], add=True)
                return _
            lax.fori_loop(0, NCH, body, None)
            remap(NCH * 128, TAILK, locbt)
            pltpu.sync_copy(ones.at[pl.ds(0, TAILK)], o_h.at[locbt], add=True)
            plsc.subcore_barrier()

    return k(dst1, dst2)


def _agg_sc(u1, u2, src1, dst1, src2, dst2):
    @functools.partial(
        pl.kernel,
        out_type=[jax.ShapeDtypeStruct((NPAD, FD), jnp.float32)] * 2,
        mesh=_mesh(),
        scratch_types=[
            pltpu.VMEM((ES,), jnp.int32),
            pltpu.VMEM((ES,), jnp.int32),
            pltpu.VMEM((128,), jnp.int32),
            pltpu.VMEM((TAILK,), jnp.int32),
            pltpu.VMEM((128, FD), jnp.float32),
            pltpu.VMEM((TAILK, FD), jnp.float32),
            pltpu.SemaphoreType.DMA,
        ],
    )
    def k(u1_h, u2_h, s1_h, d1_h, s2_h, d2_h, o1_h, o2_h,
          sbuf, dbuf, locb, locbt, rows, rowst, sem):
        cid = lax.axis_index("c")
        sid = lax.axis_index("s")
        base = cid * HALF

        for g in range(2):
            u_h = (u1_h, u2_h)[g]
            s_h = (s1_h, s2_h)[g]
            d_h = (d1_h, d2_h)[g]
            o_h = (o1_h, o2_h)[g]

            for t, cnt in ((0, 128), (128, 128), (256, 56)):
                r0 = base + sid * 312 + t
                pltpu.sync_copy(u_h.at[pl.ds(r0, cnt)],
                                rows.at[pl.ds(0, cnt)])
                pltpu.sync_copy(rows.at[pl.ds(0, cnt)],
                                o_h.at[pl.ds(r0, cnt)])

            @pl.when(sid == 0)
            def _():
                pltpu.sync_copy(u_h.at[pl.ds(base + 4992, 8)],
                                rows.at[pl.ds(0, 8)])
                pltpu.sync_copy(rows.at[pl.ds(0, 8)],
                                o_h.at[pl.ds(base + 4992, 8)])
            plsc.subcore_barrier()

            eoff = pl.multiple_of(sid * ES, 8)
            pltpu.sync_copy(s_h.at[pl.ds(eoff, ES)], sbuf)
            pltpu.sync_copy(d_h.at[pl.ds(eoff, ES)], dbuf)

            def remap(coff, kk, lb):
                for j in range(kk // 16):
                    v = dbuf[pl.ds(coff + j * 16, 16)]
                    ok = (v >= base) & (v < base + HALF)
                    lb[pl.ds(j * 16, 16)] = jnp.where(ok, v, N)

            def body(i, _):
                coff = i * 128
                remap(coff, 128, locb)
                pltpu.async_copy(u_h.at[sbuf.at[pl.ds(coff, 128)]],
                                 rows, sem).wait()
                pltpu.sync_copy(rows, o_h.at[locb], add=True)
                return _
            lax.fori_loop(0, NCH, body, None)
            remap(NCH * 128, TAILK, locbt)
            pltpu.async_copy(u_h.at[sbuf.at[pl.ds(NCH * 128, TAILK)]],
                             rowst, sem).wait()
            pltpu.sync_copy(rowst, o_h.at[locbt], add=True)
            plsc.subcore_barrier()

    return k(u1, u2, src1, dst1, src2, dst2)


def _match_sc(l_idx, r_idx, g1e, g2e, x1m, x2m, x1mm, x2mm):
    @functools.partial(
        pl.kernel,
        out_type=[jax.ShapeDtypeStruct((PN * BT, FD), jnp.float32)] * 4,
        mesh=_mesh(),
        scratch_types=[
            pltpu.VMEM((16,), jnp.int32),
            pltpu.VMEM((16, 128), jnp.int32),
            pltpu.VMEM((4, 128), jnp.int32),
            pltpu.VMEM((128, FD), jnp.float32),
            pltpu.SemaphoreType.DMA,
        ],
    )
    def k(l_h, r_h, g1_h, g2_h, x1m_h, x2m_h, x1mm_h, x2mm_h,
          oa, oax, ob, obx, aidx, nei, flat, rows, sem):
        cid = lax.axis_index("c")
        sid = lax.axis_index("s")
        wid = sid * NC + cid

        def side(idx_h, ge_h, t1_h, t2_h, o1_h, o2_h):
            off = pl.multiple_of(wid * 16, 8)
            pltpu.sync_copy(idx_h.at[pl.ds(off, 16)], aidx)
            pltpu.async_copy(ge_h.at[aidx], nei, sem).wait()
            for j in range(16):
                for h in range(2):
                    flat[j // 4, pl.ds((j % 4) * 32 + h * 16, 16)] = (
                        nei[j, pl.ds(h * 16, 16)])
            for q in range(4):
                ro = pl.multiple_of(wid * 512 + q * 128, 8)
                pltpu.async_copy(t1_h.at[flat.at[q]], rows, sem).wait()
                pltpu.sync_copy(rows, o1_h.at[pl.ds(ro, 128)])
                pltpu.async_copy(t2_h.at[flat.at[q]], rows, sem).wait()
                pltpu.sync_copy(rows, o2_h.at[pl.ds(ro, 128)])

        side(l_h, g1_h, x1m_h, x1mm_h, oa, oax)
        side(r_h, g2_h, x2m_h, x2mm_h, ob, obx)

    return k(l_idx, r_idx, g1e, g2e, x1m, x2m, x1mm, x2mm)


_R = 2000


def _tc_u(deg, x):
    def body(dg_ref, x_ref, u_ref):
        dinv = lax.rsqrt(dg_ref[:, 0:1] + 1.0)
        u_ref[...] = x_ref[...] * dinv

    return pl.pallas_call(
        body,
        grid=(N // _R,),
        in_specs=[pl.BlockSpec((_R, 8), lambda i: (i, 0)),
                  pl.BlockSpec((_R, FD), lambda i: (i, 0))],
        out_specs=pl.BlockSpec((_R, FD), lambda i: (i, 0)),
        out_shape=jax.ShapeDtypeStruct((N, FD), jnp.float32),
    )(deg, x)


def _tc_mid(S, deg, W1, b1, W2):
    def body(dg_ref, s_ref, w1_ref, b1_ref, w2_ref, o_ref):
        dinv = lax.rsqrt(dg_ref[:, 0:1] + 1.0)
        a = s_ref[...] * dinv
        h = jnp.dot(a, w1_ref[...], preferred_element_type=jnp.float32)
        h = jnp.maximum(h + b1_ref[...], 0.0)
        m = jnp.dot(h, w2_ref[...], preferred_element_type=jnp.float32)
        o_ref[...] = m * dinv

    return pl.pallas_call(
        body,
        grid=(N // _R,),
        in_specs=[pl.BlockSpec((_R, 8), lambda i: (i, 0)),
                  pl.BlockSpec((_R, FD), lambda i: (i, 0)),
                  pl.BlockSpec((FD, D1), lambda i: (0, 0)),
                  pl.BlockSpec((1, D1), lambda i: (0, 0)),
                  pl.BlockSpec((D1, D2), lambda i: (0, 0))],
        out_specs=pl.BlockSpec((_R, D2), lambda i: (i, 0)),
        out_shape=jax.ShapeDtypeStruct((N, D2), jnp.float32),
    )(deg, S, W1, b1, W2)


def _tc_norm(S, deg, b2):
    def body(dg_ref, s_ref, b2_ref, o_ref):
        dinv = lax.rsqrt(dg_ref[:, 0:1] + 1.0)
        x = s_ref[...] * dinv + b2_ref[...]
        nn = jnp.sum(x * x, axis=1, keepdims=True)
        o_ref[...] = x * lax.rsqrt(nn)

    return pl.pallas_call(
        body,
        grid=(N // _R,),
        in_specs=[pl.BlockSpec((_R, 8), lambda i: (i, 0)),
                  pl.BlockSpec((_R, D2), lambda i: (i, 0)),
                  pl.BlockSpec((1, D2), lambda i: (0, 0))],
        out_specs=pl.BlockSpec((_R, D2), lambda i: (i, 0)),
        out_shape=jax.ShapeDtypeStruct((N, D2), jnp.float32),
    )(deg, S, b2)


def _tc_sim(s2, am, ax, bm, bx):
    AB = 16

    def body(s_ref, am_ref, ax_ref, bm_ref, bx_ref, o_ref):
        s2v = s_ref[0, 0]
        dn = (((1,), (1,)), ((), ()))

        def one(j, carry):
            o32 = pl.multiple_of(j * BT, BT)
            a1 = am_ref[pl.ds(o32, BT), :]
            a2 = ax_ref[pl.ds(o32, BT), :]
            b1 = bm_ref[pl.ds(o32, BT), :]
            b2 = bx_ref[pl.ds(o32, BT), :]
            G = (lax.dot_general(a1, b1, dn, preferred_element_type=jnp.float32)
                 + s2v * lax.dot_general(a2, b2, dn,
                                         preferred_element_type=jnp.float32))
            na = (jnp.sum(a1 * a1, axis=1, keepdims=True)
                  + s2v * jnp.sum(a2 * a2, axis=1, keepdims=True))
            nb = (jnp.sum(b1 * b1, axis=1, keepdims=True)
                  + s2v * jnp.sum(b2 * b2, axis=1, keepdims=True))
            d2 = na + nb.T - 2.0 * G
            dist = jnp.sqrt(jnp.maximum(d2, 0.0) + 1e-10)
            K = jnp.exp(dist * -2.0)
            for _ in range(5):
                K = K / (jnp.sum(K, axis=1, keepdims=True) * 32.0 + 1e-10)
                K = K / (jnp.sum(K, axis=0, keepdims=True) * 32.0 + 1e-10)
            sim = jnp.sum(dist * K)
            lane = lax.broadcasted_iota(jnp.int32, (1, AB), 1)
            return jnp.where(lane == j, sim, carry)

        o_ref[0] = lax.fori_loop(0, AB, one, jnp.zeros((1, AB), jnp.float32))

    return pl.pallas_call(
        body,
        grid=(PN // AB,),
        in_specs=[pl.BlockSpec((1, 1), lambda i: (0, 0)),
                  pl.BlockSpec((AB * BT, FD), lambda i: (i, 0)),
                  pl.BlockSpec((AB * BT, FD), lambda i: (i, 0)),
                  pl.BlockSpec((AB * BT, FD), lambda i: (i, 0)),
                  pl.BlockSpec((AB * BT, FD), lambda i: (i, 0))],
        out_specs=pl.BlockSpec((1, 1, AB), lambda i: (i, 0, 0)),
        out_shape=jax.ShapeDtypeStruct((PN // AB, 1, AB), jnp.float32),
    )(s2, am, ax, bm, bx)


def _ref_rest(l_edge_index, r_edge_index, x1_input, x2_input, edge1, edge2,
              graph1_e, graph2_e, onlyGCN, graphxcom, graph1x, graph2x,
              W1, b1, W2, b2, deg1, deg2):
    def conv_pair(x, edge, deg):
        dinv = 1.0 / jnp.sqrt(deg)
        n = x.shape[0]
        sl = jnp.arange(n)
        src = jnp.concatenate([edge[0], sl])
        dst = jnp.concatenate([edge[1], sl])
        norm = dinv[src] * dinv[dst]
        h = x @ W1
        out = jnp.zeros_like(h).at[dst].add(norm[:, None] * h[src])
        x1 = jax.nn.relu(out + b1)
        h2 = x1 @ W2
        out2 = jnp.zeros_like(h2).at[dst].add(norm[:, None] * h2[src])
        x2 = out2 + b2
        return x2 / jnp.linalg.norm(x2, axis=1, keepdims=True)

    x1mid = conv_pair(x1_input, edge1, deg1)
    x2mid = conv_pair(x2_input, edge2, deg2)
    neia = graph1_e[l_edge_index].reshape(-1)
    neib = graph2_e[r_edge_index].reshape(-1)
    Pn = l_edge_index.shape[0]
    x1mm = jnp.concatenate([graphxcom, graph1x], axis=0)
    x2mm = jnp.concatenate([graphxcom, graph2x], axis=0)
    ue = jnp.where(onlyGCN != 0, 0.0, 1.0).astype(x1mid.dtype)
    axs = jnp.concatenate([x1mid[neia].reshape(Pn, BT, -1),
                           ue * x1mm[neia].reshape(Pn, BT, -1)], axis=2)
    bxs = jnp.concatenate([x2mid[neib].reshape(Pn, BT, -1),
                           ue * x2mm[neib].reshape(Pn, BT, -1)], axis=2)
    n1 = jnp.sum(axs * axs, axis=2, keepdims=True)
    n2 = jnp.sum(bxs * bxs, axis=2, keepdims=True)
    d2 = n1 + jnp.swapaxes(n2, 1, 2) - 2.0 * jnp.matmul(axs, jnp.swapaxes(bxs, 1, 2))
    dist = jnp.sqrt(jax.nn.relu(d2) + 1e-10)
    pmat = jnp.exp(-dist / 0.5)
    for _ in range(5):
        pmat = pmat / (jnp.sum(pmat, axis=2, keepdims=True) * 32.0 + 1e-10)
        pmat = pmat / (jnp.sum(pmat, axis=1, keepdims=True) * 32.0 + 1e-10)
    sims = jnp.sum(jnp.sum(dist * pmat, axis=2), axis=1)
    return (sims, x1mid, x2mid)


def kernel(l_edge_index, r_edge_index, x1_input, x2_input, edge1, edge2,
           graph1_e, graph2_e, train_size, onlyGCN,
           graphxcom, graph1x, graph2x, W1, b1, W2, b2):
    del train_size
    deg1p, deg2p = _deg_sc(edge1[1], edge2[1])
    deg1 = deg1p[:N, 0] + 1.0
    deg2 = deg2p[:N, 0] + 1.0
    return _ref_rest(l_edge_index, r_edge_index, x1_input, x2_input, edge1,
                     edge2, graph1_e, graph2_e, onlyGCN, graphxcom, graph1x,
                     graph2x, W1, b1, W2, b2, deg1, deg2)


def _unused_kernel(l_edge_index, r_edge_index, x1_input, x2_input, edge1, edge2,
           graph1_e, graph2_e, train_size, onlyGCN,
           graphxcom, graph1x, graph2x, W1, b1, W2, b2):
    del train_size
    src1, dst1 = edge1[0], edge1[1]
    src2, dst2 = edge2[0], edge2[1]

    deg1p, deg2p = _deg_sc(dst1, dst2)
    deg1 = deg1p[:N, :8]
    deg2 = deg2p[:N, :8]

    u1 = _tc_u(deg1, x1_input)
    u2 = _tc_u(deg2, x2_input)
    S1a, S2a = _agg_sc(u1, u2, src1, dst1, src2, dst2)

    b1_2d = b1.reshape(1, D1)
    b2_2d = b2.reshape(1, D2)
    up1 = _tc_mid(S1a[:N], deg1, W1, b1_2d, W2)
    up2 = _tc_mid(S2a[:N], deg2, W1, b1_2d, W2)
    S1b, S2b = _agg_sc(up1, up2, src1, dst1, src2, dst2)

    x1mid = _tc_norm(S1b[:N], deg1, b2_2d)
    x2mid = _tc_norm(S2b[:N], deg2, b2_2d)

    gpad = jnp.zeros((N, 128 - BT), jnp.int32)
    g1e_p = jnp.concatenate([graph1_e, gpad], axis=1)
    g2e_p = jnp.concatenate([graph2_e, gpad], axis=1)
    x1mm = jnp.concatenate([graphxcom, graph1x], axis=0)
    x2mm = jnp.concatenate([graphxcom, graph2x], axis=0)
    am, ax_, bm, bx_ = _match_sc(l_edge_index, r_edge_index,
                                 g1e_p, g2e_p,
                                 x1mid, x2mid, x1mm, x2mm)

    use_extra = jnp.where(onlyGCN != 0, 0.0, 1.0).astype(jnp.float32)
    s2 = (use_extra * use_extra).reshape(1, 1)
    sims = _tc_sim(s2, am, ax_, bm, bx_).reshape(-1)
    return (sims, x1mid, x2mid)


# speedup vs baseline: 7.2690x; 7.2690x over previous
import functools

import jax
import jax.numpy as jnp
from jax import lax
from jax.experimental import pallas as pl
from jax.experimental.pallas import tpu as pltpu
from jax.experimental.pallas import tpu_sc as plsc

N = 10000
FD = 256
D1 = 512
D2 = 256
E = 160000
BT = 32
PN = 512

NC = 2
NS = 16
NW = NC * NS


def _mesh():
    return plsc.VectorSubcoreMesh(core_axis_name="c", subcore_axis_name="s")


WIN = 320
NPAD2 = NW * WIN
KCH = 80


def _ptr_sc(keys1, keys2):
    @functools.partial(
        pl.kernel,
        out_type=[jax.ShapeDtypeStruct((NW, 8, 16), jnp.int32)] * 2
                 + [jax.ShapeDtypeStruct((NPAD2, 16), jnp.float32)] * 2,
        mesh=_mesh(),
        scratch_types=[
            pltpu.VMEM((24,), jnp.int32),
            pltpu.VMEM((KCH + 16,), jnp.int32),
            pltpu.VMEM((8, 16), jnp.int32),
            pltpu.VMEM((WIN, 16), jnp.float32),
        ],
    )
    def k(k1_h, k2_h, p1_h, p2_h, d1_h, d2_h, probe, kbuf, pv, dcnt):
        cid = lax.axis_index("c")
        sid = lax.axis_index("s")
        wid = sid * NC + cid
        w0 = pl.multiple_of(wid * WIN, 8)
        iota = lax.iota(jnp.int32, 16)
        one16 = jnp.full((16,), 1.0, jnp.float32)

        for g in range(2):
            k_h = (k1_h, k2_h)[g]
            p_h = (p1_h, p2_h)[g]
            d_h = (d1_h, d2_h)[g]

            def lower_bound(target):
                def bs(_, carry):
                    lo_s, hi_s = carry
                    mid = lax.div(lo_s + hi_s, jnp.int32(2))
                    mid8 = pl.multiple_of(mid & jnp.int32(-8), 8)
                    pltpu.sync_copy(k_h.at[pl.ds(mid8, 24)], probe)
                    v = probe[pl.ds(mid - mid8, 16)][0]
                    pred = v >= target
                    return (jnp.where(pred, lo_s, mid + 1),
                            jnp.where(pred, mid, hi_s))
                lo_s, hi_s = lax.fori_loop(0, 18, bs, (jnp.int32(0), jnp.int32(E)))
                return hi_s

            lo = lower_bound(w0 * 16384)
            hi = lower_bound((w0 + WIN) * 16384)
            pv[0, pl.ds(0, 16)] = jnp.where(iota == 0, lo, 0)
            pv[1, pl.ds(0, 16)] = jnp.where(iota == 0, hi, 0)
            for r in range(2, 8):
                pv[r, pl.ds(0, 16)] = jnp.zeros((16,), jnp.int32)
            pltpu.sync_copy(pv, p_h.at[wid])

            def zr(t, _):
                dcnt[t, pl.ds(0, 16)] = jnp.zeros((16,), jnp.float32)
                return _
            lax.fori_loop(0, WIN, zr, None)

            lo8 = lo & jnp.int32(-8)
            nch = lax.div(hi - lo8 + KCH - 1, jnp.int32(KCH))

            def chunk(i, _):
                off = pl.multiple_of(lo8 + i * KCH, 8)
                pltpu.sync_copy(k_h.at[pl.ds(off, KCH)], kbuf.at[pl.ds(0, KCH)])
                start = jnp.where(i == 0, lo - lo8, 0)
                stop = jnp.minimum(hi - off, KCH)

                def body(e, _):
                    loc = lax.shift_right_logical(kbuf[pl.ds(e, 16)][0], 14) - w0
                    dcnt[loc, pl.ds(0, 16)] = dcnt[loc, pl.ds(0, 16)] + one16
                    return _
                lax.fori_loop(start, stop, body, None)
                return _
            lax.fori_loop(0, nch, chunk, None)
            pltpu.sync_copy(dcnt, d_h.at[pl.ds(w0, WIN)])

    return k(keys1, keys2)


def _agg_sc(u1, u2, keys1, ptr1, keys2, ptr2):
    @functools.partial(
        pl.kernel,
        out_type=[jax.ShapeDtypeStruct((NPAD2, FD), jnp.float32)] * 2,
        mesh=_mesh(),
        scratch_types=[
            pltpu.VMEM((KCH + 16,), jnp.int32),
            pltpu.VMEM((KCH + 16,), jnp.int32),
            pltpu.VMEM((KCH,), jnp.int32),
            pltpu.VMEM((KCH,), jnp.int32),
            pltpu.VMEM((8, 16), jnp.int32),
            pltpu.VMEM((KCH, FD), jnp.float32),
            pltpu.VMEM((KCH, FD), jnp.float32),
            pltpu.VMEM((WIN, FD), jnp.float32),
            pltpu.SemaphoreType.DMA,
            pltpu.SemaphoreType.DMA,
        ],
    )
    def k(u1_h, u2_h, k1_h, p1_h, k2_h, p2_h, o1_h, o2_h,
          kbufa, kbufb, gbufa, gbufb, pv, rowsa, rowsb, acc,
          sema, semb):
        cid = lax.axis_index("c")
        sid = lax.axis_index("s")
        wid = sid * NC + cid
        w0 = pl.multiple_of(wid * WIN, 8)

        for g in range(2):
            u_h = (u1_h, u2_h)[g]
            k_h = (k1_h, k2_h)[g]
            p_h = (p1_h, p2_h)[g]
            o_h = (o1_h, o2_h)[g]

            pltpu.sync_copy(u_h.at[pl.ds(w0, WIN)], acc)

            pltpu.sync_copy(p_h.at[wid], pv)
            lo = pv[0, pl.ds(0, 16)][0]
            hi = pv[1, pl.ds(0, 16)][0]
            lo8 = lo & jnp.int32(-8)
            nch = lax.div(hi - lo8 + KCH - 1, jnp.int32(KCH))

            def stage(i, kb, gb, rv, sem):
                off = pl.multiple_of(lo8 + i * KCH, 8)
                pltpu.sync_copy(k_h.at[pl.ds(off, KCH)], kb.at[pl.ds(0, KCH)])
                for j in range(KCH // 16):
                    gb[pl.ds(j * 16, 16)] = kb[pl.ds(j * 16, 16)] & 16383
                pltpu.async_copy(u_h.at[gb], rv, sem)

            def rmw(i, kb, gb, rv, sem):
                pltpu.make_async_copy(u_h.at[gb], rv, sem).wait()
                off = lo8 + i * KCH
                start = jnp.where(i == 0, lo - lo8, 0)
                stop = jnp.minimum(hi - off, KCH)

                def body(e, _):
                    kk = kb[pl.ds(e, 16)][0]
                    loc = lax.shift_right_logical(kk, 14) - w0
                    for h in range(FD // 16):
                        acc[loc, pl.ds(h * 16, 16)] = (
                            acc[loc, pl.ds(h * 16, 16)]
                            + rv[e, pl.ds(h * 16, 16)])
                    return _
                lax.fori_loop(start, stop, body, None)

            @pl.when(nch > 0)
            def _():
                stage(0, kbufa, gbufa, rowsa, sema)

            def pair(pp, _):
                i0 = 2 * pp
                i1 = i0 + 1

                @pl.when(i1 < nch)
                def _():
                    stage(i1, kbufb, gbufb, rowsb, semb)
                rmw(i0, kbufa, gbufa, rowsa, sema)

                @pl.when(i1 < nch)
                def _():
                    @pl.when(i1 + 1 < nch)
                    def _():
                        stage(i1 + 1, kbufa, gbufa, rowsa, sema)
                    rmw(i1, kbufb, gbufb, rowsb, semb)
                return _
            lax.fori_loop(0, lax.div(nch + 1, jnp.int32(2)), pair, None)

            pltpu.sync_copy(acc, o_h.at[pl.ds(w0, WIN)])

    return k(u1, u2, keys1, ptr1, keys2, ptr2)


def _match_sc(l_idx, r_idx, g1e, g2e, x1m, x2m, x1mm, x2mm):
    @functools.partial(
        pl.kernel,
        out_type=[jax.ShapeDtypeStruct((PN * BT, FD), jnp.float32)] * 4,
        mesh=_mesh(),
        scratch_types=[
            pltpu.VMEM((16,), jnp.int32),
            pltpu.VMEM((16, 128), jnp.int32),
            pltpu.VMEM((4, 128), jnp.int32),
            pltpu.VMEM((128, FD), jnp.float32),
            pltpu.SemaphoreType.DMA,
        ],
    )
    def k(l_h, r_h, g1_h, g2_h, x1m_h, x2m_h, x1mm_h, x2mm_h,
          oa, oax, ob, obx, aidx, nei, flat, rows, sem):
        cid = lax.axis_index("c")
        sid = lax.axis_index("s")
        wid = sid * NC + cid

        def side(idx_h, ge_h, t1_h, t2_h, o1_h, o2_h):
            off = pl.multiple_of(wid * 16, 8)
            pltpu.sync_copy(idx_h.at[pl.ds(off, 16)], aidx)
            pltpu.async_copy(ge_h.at[aidx], nei, sem).wait()
            for j in range(16):
                for h in range(2):
                    flat[j // 4, pl.ds((j % 4) * 32 + h * 16, 16)] = (
                        nei[j, pl.ds(h * 16, 16)])
            for q in range(4):
                ro = pl.multiple_of(wid * 512 + q * 128, 8)
                pltpu.async_copy(t1_h.at[flat.at[q]], rows, sem).wait()
                pltpu.sync_copy(rows, o1_h.at[pl.ds(ro, 128)])
                pltpu.async_copy(t2_h.at[flat.at[q]], rows, sem).wait()
                pltpu.sync_copy(rows, o2_h.at[pl.ds(ro, 128)])

        side(l_h, g1_h, x1m_h, x1mm_h, oa, oax)
        side(r_h, g2_h, x2m_h, x2mm_h, ob, obx)

    return k(l_idx, r_idx, g1e, g2e, x1m, x2m, x1mm, x2mm)


_R = 2000


def _tc_u(deg, x):
    def body(dg_ref, x_ref, u_ref):
        dinv = lax.rsqrt(dg_ref[:, 0:1] + 1.0)
        u_ref[...] = x_ref[...] * dinv

    return pl.pallas_call(
        body,
        grid=(N // _R,),
        in_specs=[pl.BlockSpec((_R, 8), lambda i: (i, 0)),
                  pl.BlockSpec((_R, FD), lambda i: (i, 0))],
        out_specs=pl.BlockSpec((_R, FD), lambda i: (i, 0)),
        out_shape=jax.ShapeDtypeStruct((N, FD), jnp.float32),
    )(deg, x)


def _tc_mid(S, deg, W1, b1, W2):
    def body(dg_ref, s_ref, w1_ref, b1_ref, w2_ref, o_ref):
        dinv = lax.rsqrt(dg_ref[:, 0:1] + 1.0)
        a = s_ref[...] * dinv
        h = jnp.dot(a, w1_ref[...], preferred_element_type=jnp.float32)
        h = jnp.maximum(h + b1_ref[...], 0.0)
        m = jnp.dot(h, w2_ref[...], preferred_element_type=jnp.float32)
        o_ref[...] = m * dinv

    return pl.pallas_call(
        body,
        grid=(N // _R,),
        in_specs=[pl.BlockSpec((_R, 8), lambda i: (i, 0)),
                  pl.BlockSpec((_R, FD), lambda i: (i, 0)),
                  pl.BlockSpec((FD, D1), lambda i: (0, 0)),
                  pl.BlockSpec((1, D1), lambda i: (0, 0)),
                  pl.BlockSpec((D1, D2), lambda i: (0, 0))],
        out_specs=pl.BlockSpec((_R, D2), lambda i: (i, 0)),
        out_shape=jax.ShapeDtypeStruct((N, D2), jnp.float32),
    )(deg, S, W1, b1, W2)


def _tc_norm(S, deg, b2):
    def body(dg_ref, s_ref, b2_ref, o_ref):
        dinv = lax.rsqrt(dg_ref[:, 0:1] + 1.0)
        x = s_ref[...] * dinv + b2_ref[...]
        nn = jnp.sum(x * x, axis=1, keepdims=True)
        o_ref[...] = x * lax.rsqrt(nn)

    return pl.pallas_call(
        body,
        grid=(N // _R,),
        in_specs=[pl.BlockSpec((_R, 8), lambda i: (i, 0)),
                  pl.BlockSpec((_R, D2), lambda i: (i, 0)),
                  pl.BlockSpec((1, D2), lambda i: (0, 0))],
        out_specs=pl.BlockSpec((_R, D2), lambda i: (i, 0)),
        out_shape=jax.ShapeDtypeStruct((N, D2), jnp.float32),
    )(deg, S, b2)


def _tc_sim(s2, am, ax, bm, bx):
    AB = 16

    def body(s_ref, am_ref, ax_ref, bm_ref, bx_ref, o_ref):
        s2v = s_ref[0, 0]
        dn = (((1,), (1,)), ((), ()))

        def one(j, carry):
            o32 = pl.multiple_of(j * BT, BT)
            a1 = am_ref[pl.ds(o32, BT), :]
            a2 = ax_ref[pl.ds(o32, BT), :]
            b1 = bm_ref[pl.ds(o32, BT), :]
            b2 = bx_ref[pl.ds(o32, BT), :]
            G = (lax.dot_general(a1, b1, dn, preferred_element_type=jnp.float32)
                 + s2v * lax.dot_general(a2, b2, dn,
                                         preferred_element_type=jnp.float32))
            na = (jnp.sum(a1 * a1, axis=1, keepdims=True)
                  + s2v * jnp.sum(a2 * a2, axis=1, keepdims=True))
            nb = (jnp.sum(b1 * b1, axis=1, keepdims=True)
                  + s2v * jnp.sum(b2 * b2, axis=1, keepdims=True))
            d2 = na + nb.T - 2.0 * G
            dist = jnp.sqrt(jnp.maximum(d2, 0.0) + 1e-10)
            K = jnp.exp(dist * -2.0)
            for _ in range(5):
                K = K / (jnp.sum(K, axis=1, keepdims=True) * 32.0 + 1e-10)
                K = K / (jnp.sum(K, axis=0, keepdims=True) * 32.0 + 1e-10)
            sim = jnp.sum(dist * K)
            lane = lax.broadcasted_iota(jnp.int32, (1, AB), 1)
            return jnp.where(lane == j, sim, carry)

        o_ref[0] = lax.fori_loop(0, AB, one, jnp.zeros((1, AB), jnp.float32))

    return pl.pallas_call(
        body,
        grid=(PN // AB,),
        in_specs=[pl.BlockSpec((1, 1), lambda i: (0, 0)),
                  pl.BlockSpec((AB * BT, FD), lambda i: (i, 0)),
                  pl.BlockSpec((AB * BT, FD), lambda i: (i, 0)),
                  pl.BlockSpec((AB * BT, FD), lambda i: (i, 0)),
                  pl.BlockSpec((AB * BT, FD), lambda i: (i, 0))],
        out_specs=pl.BlockSpec((1, 1, AB), lambda i: (i, 0, 0)),
        out_shape=jax.ShapeDtypeStruct((PN // AB, 1, AB), jnp.float32),
    )(s2, am, ax, bm, bx)


def _prep_graph(edge):
    src, dst = edge[0], edge[1]
    keys = jnp.sort(dst * 16384 + src, stable=False)
    return jnp.concatenate([keys, jnp.zeros((136,), jnp.int32)])


def kernel(l_edge_index, r_edge_index, x1_input, x2_input, edge1, edge2,
           graph1_e, graph2_e, train_size, onlyGCN,
           graphxcom, graph1x, graph2x, W1, b1, W2, b2):
    del train_size
    keys1 = _prep_graph(edge1)
    keys2 = _prep_graph(edge2)
    ptr1, ptr2, deg1p, deg2p = _ptr_sc(keys1, keys2)
    deg1 = deg1p[:N, :8]
    deg2 = deg2p[:N, :8]

    u1 = _tc_u(deg1, x1_input)
    u2 = _tc_u(deg2, x2_input)
    zpad = jnp.zeros((NPAD2 - N, FD), jnp.float32)
    S1a, S2a = _agg_sc(jnp.concatenate([u1, zpad], 0),
                       jnp.concatenate([u2, zpad], 0),
                       keys1, ptr1, keys2, ptr2)

    b1_2d = b1.reshape(1, D1)
    b2_2d = b2.reshape(1, D2)
    up1 = _tc_mid(S1a[:N], deg1, W1, b1_2d, W2)
    up2 = _tc_mid(S2a[:N], deg2, W1, b1_2d, W2)
    S1b, S2b = _agg_sc(jnp.concatenate([up1, zpad], 0),
                       jnp.concatenate([up2, zpad], 0),
                       keys1, ptr1, keys2, ptr2)

    x1mid = _tc_norm(S1b[:N], deg1, b2_2d)
    x2mid = _tc_norm(S2b[:N], deg2, b2_2d)

    gpad = jnp.zeros((N, 128 - BT), jnp.int32)
    g1e_p = jnp.concatenate([graph1_e, gpad], axis=1)
    g2e_p = jnp.concatenate([graph2_e, gpad], axis=1)
    x1mm = jnp.concatenate([graphxcom, graph1x], axis=0)
    x2mm = jnp.concatenate([graphxcom, graph2x], axis=0)
    am, ax_, bm, bx_ = _match_sc(l_edge_index, r_edge_index,
                                 g1e_p, g2e_p,
                                 x1mid, x2mid, x1mm, x2mm)

    use_extra = jnp.where(onlyGCN != 0, 0.0, 1.0).astype(jnp.float32)
    s2 = (use_extra * use_extra).reshape(1, 1)
    sims = _tc_sim(s2, am, ax_, bm, bx_).reshape(-1)
    return (sims, x1mid, x2mid)
